# baseline (speedup 1.0000x reference)
import functools

import jax
import jax.numpy as jnp
from jax import lax
from jax.experimental import pallas as pl
from jax.experimental.pallas import tpu as pltpu


def _tc_cumsum(x, s_blk=256, f_blk=1024, interpret=False):
    b, s, f = x.shape
    sb, fb = s // s_blk, f // f_blk

    def body(x_ref, o_ref, carry_ref):
        si = pl.program_id(1)

        @pl.when(si == 0)
        def _():
            carry_ref[...] = jnp.zeros_like(carry_ref)

        blk = x_ref[0]
        r = lax.broadcasted_iota(jnp.int32, (s_blk, s_blk), 0)
        c = lax.broadcasted_iota(jnp.int32, (s_blk, s_blk), 1)
        tri = jnp.where(r >= c, 1.0, 0.0)
        out = (
            jax.lax.dot(tri, blk, precision=jax.lax.Precision.HIGHEST)
            + carry_ref[...]
        )
        o_ref[0] = out
        carry_ref[...] = out[s_blk - 1 :, :]

    return pl.pallas_call(
        body,
        grid=(b * fb, sb),
        in_specs=[
            pl.BlockSpec(
                (1, s_blk, f_blk), lambda bf, si: (bf // fb, si, bf % fb)
            )
        ],
        out_specs=pl.BlockSpec(
            (1, s_blk, f_blk), lambda bf, si: (bf // fb, si, bf % fb)
        ),
        out_shape=jax.ShapeDtypeStruct((b, s, f), jnp.float32),
        scratch_shapes=[pltpu.VMEM((1, f_blk), jnp.float32)],
        compiler_params=pltpu.CompilerParams(
            dimension_semantics=("arbitrary", "arbitrary")
        ),
        interpret=interpret,
    )(x)


def kernel(x, dim):
    del dim
    return _tc_cumsum(x)


# speedup vs baseline: 1.2848x; 1.2848x over previous
import functools

import jax
import jax.numpy as jnp
from jax import lax
from jax.experimental import pallas as pl
from jax.experimental.pallas import tpu as pltpu


def _tc_cumsum(x, s_blk=256, f_blk=1024, interpret=False):
    b, s, f = x.shape
    sb, fb = s // s_blk, f // f_blk

    def body(x_ref, o_ref, carry_ref):
        si = pl.program_id(1)

        @pl.when(si == 0)
        def _():
            carry_ref[...] = jnp.zeros_like(carry_ref)

        blk = x_ref[0]
        r = lax.broadcasted_iota(jnp.int32, (s_blk, s_blk), 0)
        c = lax.broadcasted_iota(jnp.int32, (s_blk, s_blk), 1)
        tri = jnp.where(r >= c, 1.0, 0.0)
        out = (
            jax.lax.dot(tri, blk, precision=jax.lax.Precision.DEFAULT)
            + carry_ref[...]
        )
        o_ref[0] = out
        carry_ref[...] = out[s_blk - 1 :, :]

    return pl.pallas_call(
        body,
        grid=(b * fb, sb),
        in_specs=[
            pl.BlockSpec(
                (1, s_blk, f_blk), lambda bf, si: (bf // fb, si, bf % fb)
            )
        ],
        out_specs=pl.BlockSpec(
            (1, s_blk, f_blk), lambda bf, si: (bf // fb, si, bf % fb)
        ),
        out_shape=jax.ShapeDtypeStruct((b, s, f), jnp.float32),
        scratch_shapes=[pltpu.VMEM((1, f_blk), jnp.float32)],
        compiler_params=pltpu.CompilerParams(
            dimension_semantics=("arbitrary", "arbitrary")
        ),
        interpret=interpret,
    )(x)


def kernel(x, dim):
    del dim
    return _tc_cumsum(x)


# speedup vs baseline: 2.0576x; 1.6016x over previous
import functools

import jax
import jax.numpy as jnp
from jax import lax
from jax.experimental import pallas as pl
from jax.experimental.pallas import tpu as pltpu


def _tc_cumsum(x, s_blk=512, f_blk=2048, interpret=False):
    b, s, f = x.shape
    sb, fb = s // s_blk, f // f_blk

    def body(x_ref, o_ref, carry_ref):
        si = pl.program_id(1)

        @pl.when(si == 0)
        def _():
            carry_ref[...] = jnp.zeros_like(carry_ref)

        blk = x_ref[0]
        r = lax.broadcasted_iota(jnp.int32, (s_blk, s_blk), 0)
        c = lax.broadcasted_iota(jnp.int32, (s_blk, s_blk), 1)
        tri = jnp.where(r >= c, 1.0, 0.0)
        out = (
            jax.lax.dot(tri, blk, precision=jax.lax.Precision.DEFAULT)
            + carry_ref[...]
        )
        o_ref[0] = out
        carry_ref[...] = out[s_blk - 1 :, :]

    return pl.pallas_call(
        body,
        grid=(b * fb, sb),
        in_specs=[
            pl.BlockSpec(
                (1, s_blk, f_blk), lambda bf, si: (bf // fb, si, bf % fb)
            )
        ],
        out_specs=pl.BlockSpec(
            (1, s_blk, f_blk), lambda bf, si: (bf // fb, si, bf % fb)
        ),
        out_shape=jax.ShapeDtypeStruct((b, s, f), jnp.float32),
        scratch_shapes=[pltpu.VMEM((1, f_blk), jnp.float32)],
        compiler_params=pltpu.CompilerParams(
            dimension_semantics=("arbitrary", "arbitrary")
        ),
        interpret=interpret,
    )(x)


def kernel(x, dim):
    del dim
    return _tc_cumsum(x)
